# baseline (device time: 26700 ns/iter reference)
import jax
import jax.numpy as jnp
from jax import lax
from jax.experimental import pallas as pl
from jax.experimental.pallas import tpu as pltpu

N_DEV = 16
NSLOTS = 4
OFFSETS = list(range(N_DEV))


def kernel(x, w_mat, scale_x, scale_w):
    m_per, k = x.shape
    _, n = w_mat.shape
    n_per = n // N_DEV
    k_half = k // 2

    def body(x_ref, w_ref, sx_ref, sw_ref, out_ref,
             x8_ref, wtile_ref, y8_ref, ybuf_ref,
             load_sems, send_sems, recv_sems):
        my_pos = lax.axis_index("i")
        scale = sx_ref[0] * sw_ref[0]

        def start_load(t):
            d = lax.rem(my_pos + OFFSETS[t], N_DEV)
            slot = t % NSLOTS
            cp = pltpu.make_async_copy(
                w_ref.at[:, pl.ds(d * n_per, n_per)],
                wtile_ref.at[slot],
                load_sems.at[slot],
            )
            cp.start()
            return cp

        loads = [start_load(t) for t in range(NSLOTS - 1)]
        x8_ref[:, :] = x_ref[:, :].astype(jnp.float8_e4m3fn)

        rdmas = []
        for t in range(N_DEV):
            if t + NSLOTS - 1 < N_DEV:
                loads.append(start_load(t + NSLOTS - 1))
            loads[t].wait()

            acc = jnp.dot(
                x8_ref[:, :],
                wtile_ref[t % NSLOTS].astype(jnp.float8_e5m2),
                preferred_element_type=jnp.float32,
            )
            y = acc * scale
            blk = jnp.clip(
                jnp.rint(acc * jax.nn.sigmoid(y)), -127.0, 127.0
            ).astype(jnp.int8)

            o = OFFSETS[t]
            if o == 0:
                ybuf_ref[pl.ds(my_pos * m_per, m_per), :] = blk
            else:
                d = lax.rem(my_pos + o, N_DEV)
                y8_ref[:, pl.ds(o * n_per, n_per)] = blk
                rdma = pltpu.make_async_remote_copy(
                    src_ref=y8_ref.at[:, pl.ds(o * n_per, n_per)],
                    dst_ref=ybuf_ref.at[pl.ds(my_pos * m_per, m_per), :],
                    send_sem=send_sems.at[o - 1],
                    recv_sem=recv_sems.at[o - 1],
                    device_id=(d,),
                    device_id_type=pl.DeviceIdType.MESH,
                )
                rdma.start()
                rdmas.append(rdma)

        for rdma in rdmas:
            rdma.wait()

        out_ref[:, :] = ybuf_ref[:, :].astype(jnp.float32) * scale

    out_shape = jax.ShapeDtypeStruct((m_per * N_DEV, n_per), jnp.float32)
    return pl.pallas_call(
        body,
        out_shape=out_shape,
        in_specs=[
            pl.BlockSpec(memory_space=pltpu.VMEM),
            pl.BlockSpec(memory_space=pltpu.MemorySpace.HBM),
            pl.BlockSpec(memory_space=pltpu.SMEM),
            pl.BlockSpec(memory_space=pltpu.SMEM),
        ],
        out_specs=pl.BlockSpec(memory_space=pltpu.VMEM),
        scratch_shapes=[
            pltpu.VMEM((m_per, k), jnp.float8_e4m3fn),
            pltpu.VMEM((NSLOTS, k, n_per), jnp.float32),
            pltpu.VMEM((m_per, n), jnp.int8),
            pltpu.VMEM((m_per * N_DEV, n_per), jnp.int8),
            pltpu.SemaphoreType.DMA((NSLOTS,)),
            pltpu.SemaphoreType.DMA((N_DEV - 1,)),
            pltpu.SemaphoreType.DMA((N_DEV - 1,)),
        ],
        compiler_params=pltpu.CompilerParams(
            vmem_limit_bytes=48 * 1024 * 1024,
        ),
    )(x, w_mat, scale_x, scale_w)


# device time: 26263 ns/iter; 1.0166x vs baseline; 1.0166x over previous
import jax
import jax.numpy as jnp
from jax import lax
from jax.experimental import pallas as pl
from jax.experimental.pallas import tpu as pltpu

N_DEV = 16
NSLOTS = 4
OFFSETS = list(range(1, N_DEV)) + [0]


def kernel(x, w_mat, scale_x, scale_w):
    m_per, k = x.shape
    _, n = w_mat.shape
    n_per = n // N_DEV
    k_half = k // 2

    def body(x_ref, w_ref, sx_ref, sw_ref, out_ref,
             x8_ref, wtile_ref, y8_ref, ybuf_ref,
             load_sems, send_sems, recv_sems):
        my_pos = lax.axis_index("i")
        scale = sx_ref[0] * sw_ref[0]

        def start_load(t):
            d = lax.rem(my_pos + OFFSETS[t], N_DEV)
            slot = t % NSLOTS
            cp = pltpu.make_async_copy(
                w_ref.at[:, pl.ds(d * n_per, n_per)],
                wtile_ref.at[slot],
                load_sems.at[slot],
            )
            cp.start()
            return cp

        loads = [start_load(t) for t in range(NSLOTS - 1)]
        x8_ref[:, :] = x_ref[:, :].astype(jnp.float8_e4m3fn)

        rdmas = []
        for t in range(N_DEV):
            if t + NSLOTS - 1 < N_DEV:
                loads.append(start_load(t + NSLOTS - 1))
            loads[t].wait()

            acc = jnp.dot(
                x8_ref[:, :],
                wtile_ref[t % NSLOTS].astype(jnp.float8_e5m2),
                preferred_element_type=jnp.float32,
            )
            y = acc * scale
            blk = jnp.clip(
                jnp.rint(acc * jax.nn.sigmoid(y)), -127.0, 127.0
            ).astype(jnp.int8)

            o = OFFSETS[t]
            if o == 0:
                ybuf_ref[pl.ds(my_pos * m_per, m_per), :] = blk
            else:
                d = lax.rem(my_pos + o, N_DEV)
                y8_ref[:, pl.ds(o * n_per, n_per)] = blk
                rdma = pltpu.make_async_remote_copy(
                    src_ref=y8_ref.at[:, pl.ds(o * n_per, n_per)],
                    dst_ref=ybuf_ref.at[pl.ds(my_pos * m_per, m_per), :],
                    send_sem=send_sems.at[o - 1],
                    recv_sem=recv_sems.at[o - 1],
                    device_id=(d,),
                    device_id_type=pl.DeviceIdType.MESH,
                )
                rdma.start()
                rdmas.append(rdma)

        for rdma in rdmas:
            rdma.wait()

        out_ref[:, :] = ybuf_ref[:, :].astype(jnp.float32) * scale

    out_shape = jax.ShapeDtypeStruct((m_per * N_DEV, n_per), jnp.float32)
    return pl.pallas_call(
        body,
        out_shape=out_shape,
        in_specs=[
            pl.BlockSpec(memory_space=pltpu.VMEM),
            pl.BlockSpec(memory_space=pltpu.MemorySpace.HBM),
            pl.BlockSpec(memory_space=pltpu.SMEM),
            pl.BlockSpec(memory_space=pltpu.SMEM),
        ],
        out_specs=pl.BlockSpec(memory_space=pltpu.VMEM),
        scratch_shapes=[
            pltpu.VMEM((m_per, k), jnp.float8_e4m3fn),
            pltpu.VMEM((NSLOTS, k, n_per), jnp.float32),
            pltpu.VMEM((m_per, n), jnp.int8),
            pltpu.VMEM((m_per * N_DEV, n_per), jnp.int8),
            pltpu.SemaphoreType.DMA((NSLOTS,)),
            pltpu.SemaphoreType.DMA((N_DEV - 1,)),
            pltpu.SemaphoreType.DMA((N_DEV - 1,)),
        ],
        compiler_params=pltpu.CompilerParams(
            vmem_limit_bytes=48 * 1024 * 1024,
        ),
    )(x, w_mat, scale_x, scale_w)
